# Initial kernel scaffold; baseline (speedup 1.0000x reference)
import jax
import jax.numpy as jnp
from jax.experimental import pallas as pl


def kernel(x, freqs_cos, freqs_sin, wq, wk, wv, wo, start_pos):
    raise NotImplementedError("write your pallas kernel here")


# speedup vs baseline: 1.8639x; 1.8639x over previous
import jax
import jax.numpy as jnp
import numpy as np
from jax.experimental import pallas as pl
from jax.experimental.pallas import tpu as pltpu

_L = 2048
_D = 1024
_HQ = 16
_HKV = 4
_HD = 64
_NREP = _HQ // _HKV
_BLK = 128
_NB = _L // _BLK
_TOPK = 8
_SCALE = 1.0 / np.sqrt(_HD)
_NEG = -1e9


def _rope_halves(z, c, s):
    a = z[:, : _HD // 2]
    b = z[:, _HD // 2:]
    return jnp.concatenate([a * c - b * s, a * s + b * c], axis=1)


def _attn_body(x_ref, cos_ref, sin_ref, wq_ref, wk_ref, wv_ref, wo_ref,
               y_ref, kc_ref, vc_ref):
    qb = pl.program_id(0)
    h = pl.program_id(1)

    xb = x_ref[...]
    c = cos_ref[...]
    s = sin_ref[...]

    @pl.when(jnp.logical_and(qb == 0, h == 0))
    def _zero_v():
        vc_ref[...] = jnp.zeros((_HKV, _L, _HD), jnp.float32)

    @pl.when(h == 0)
    def _kv():
        for g in range(_HKV):
            kg = jax.lax.dot_general(xb, wk_ref[g], (((1,), (0,)), ((), ())),
                                     preferred_element_type=jnp.float32)
            kc_ref[g, pl.ds(qb * _BLK, _BLK), :] = _rope_halves(kg, c, s)
            vg = jax.lax.dot_general(xb, wv_ref[g], (((1,), (0,)), ((), ())),
                                     preferred_element_type=jnp.float32)
            vc_ref[g, pl.ds(qb * _BLK, _BLK), :] = vg

    qh = jax.lax.dot_general(xb, wq_ref[h], (((1,), (0,)), ((), ())),
                             preferred_element_type=jnp.float32)
    qr = _rope_halves(qh, c, s)

    g = h // _NREP
    kk = kc_ref[g]
    scores = jax.lax.dot_general(qr, kk, (((1,), (1,)), ((), ())),
                                 preferred_element_type=jnp.float32)
    row = qb * _BLK + jax.lax.broadcasted_iota(jnp.int32, (_BLK, _L), 0)
    col = jax.lax.broadcasted_iota(jnp.int32, (_BLK, _L), 1)
    sc = jnp.where(col <= row, scores * _SCALE, _NEG)

    bms = [jnp.max(sc[:, j * _BLK:(j + 1) * _BLK], axis=1, keepdims=True)
           for j in range(_NB)]
    bm = jnp.concatenate(bms, axis=1)
    counts = jnp.zeros((_BLK, _NB), jnp.float32)
    for i in range(_NB):
        counts = counts + (bms[i] > bm).astype(jnp.float32)
    keep = counts < float(_TOPK)
    keep_full = jnp.concatenate(
        [jnp.broadcast_to(keep[:, j:j + 1], (_BLK, _BLK)) for j in range(_NB)],
        axis=1)
    masked = jnp.where(keep_full, sc, _NEG)
    m = jnp.max(masked, axis=1, keepdims=True)
    e = jnp.exp(masked - m)
    denom = jnp.sum(e, axis=1, keepdims=True)
    vv = vc_ref[g]
    ov = jax.lax.dot_general(e, vv, (((1,), (0,)), ((), ())),
                             preferred_element_type=jnp.float32)
    ov = ov / denom
    contrib = jax.lax.dot_general(ov, wo_ref[h], (((1,), (0,)), ((), ())),
                                  preferred_element_type=jnp.float32)

    @pl.when(h == 0)
    def _init():
        y_ref[...] = contrib

    @pl.when(h > 0)
    def _acc():
        y_ref[...] = y_ref[...] + contrib


def kernel(x, freqs_cos, freqs_sin, wq, wk, wv, wo, start_pos):
    b, l, d = x.shape
    cos = jax.lax.dynamic_slice_in_dim(freqs_cos, start_pos, l, axis=0)
    sin = jax.lax.dynamic_slice_in_dim(freqs_sin, start_pos, l, axis=0)

    i = np.arange(_HD)
    src = np.where(i < _HD // 2, 2 * i, 2 * (i - _HD // 2) + 1)
    perm_q = (np.arange(_HQ)[:, None] * _HD + src[None, :]).reshape(-1)
    perm_k = (np.arange(_HKV)[:, None] * _HD + src[None, :]).reshape(-1)
    wq3 = jnp.transpose(wq[perm_q, :].reshape(_HQ, _HD, _D), (0, 2, 1))
    wk3 = jnp.transpose(wk[perm_k, :].reshape(_HKV, _HD, _D), (0, 2, 1))
    wv3 = jnp.transpose(wv.reshape(_HKV, _HD, _D), (0, 2, 1))
    wo3 = jnp.transpose(wo.reshape(_D, _HQ, _HD), (1, 2, 0))
    x2 = x.reshape(l, d)

    y = pl.pallas_call(
        _attn_body,
        grid=(l // _BLK, _HQ),
        in_specs=[
            pl.BlockSpec((_BLK, _D), lambda qb, h: (qb, 0)),
            pl.BlockSpec((_BLK, _HD // 2), lambda qb, h: (qb, 0)),
            pl.BlockSpec((_BLK, _HD // 2), lambda qb, h: (qb, 0)),
            pl.BlockSpec((_HQ, _D, _HD), lambda qb, h: (0, 0, 0)),
            pl.BlockSpec((_HKV, _D, _HD), lambda qb, h: (0, 0, 0)),
            pl.BlockSpec((_HKV, _D, _HD), lambda qb, h: (0, 0, 0)),
            pl.BlockSpec((_HQ, _HD, _D), lambda qb, h: (0, 0, 0)),
        ],
        out_specs=pl.BlockSpec((_BLK, _D), lambda qb, h: (qb, 0)),
        out_shape=jax.ShapeDtypeStruct((l, _D), jnp.float32),
        scratch_shapes=[
            pltpu.VMEM((_HKV, _L, _HD), jnp.float32),
            pltpu.VMEM((_HKV, _L, _HD), jnp.float32),
        ],
        compiler_params=pltpu.CompilerParams(
            dimension_semantics=("arbitrary", "arbitrary")),
    )(x2, cos, sin, wq3, wk3, wv3, wo3)
    return y.reshape(b, l, _D)
